# baseline (speedup 1.0000x reference)
import functools

import jax
import jax.numpy as jnp
from jax import lax
from jax.experimental import pallas as pl
from jax.experimental.pallas import tpu as pltpu
from jax.experimental.pallas import tpu_sc as plsc

N_NODES = 10000
N_EDGES = 320000
D_IN = 128
D_HID = 32
D_OUT = 16

NC = 2
NS = 16
NW = NC * NS
B = 128
NB = N_EDGES // B
BPT = NB // NW
RPT = 624
REM = N_NODES - NS * RPT


def _edge_aggregate(h, a_src_n, a_dst_n, srcb, dstb, d):
  mesh = plsc.VectorSubcoreMesh(core_axis_name="c", subcore_axis_name="s",
                                num_cores=NC, num_subcores=NS)

  @functools.partial(
      pl.kernel,
      out_type=jax.ShapeDtypeStruct((NC, N_NODES, d + 16), jnp.float32),
      mesh=mesh,
      compiler_params=pltpu.CompilerParams(use_tc_tiling_on_sc=False,
                                           needs_layout_passes=False),
      scratch_types=[
          pltpu.VMEM((N_NODES,), jnp.float32),
          pltpu.VMEM((N_NODES,), jnp.float32),
          pltpu.VMEM((BPT + 1, B), jnp.int32),
          pltpu.VMEM((BPT + 1, B), jnp.int32),
          pltpu.VMEM((B,), jnp.float32),
          pltpu.VMEM((B,), jnp.float32),
          pltpu.VMEM((B, d), jnp.float32),
          pltpu.VMEM((B, d), jnp.float32),
          pltpu.VMEM((B, d + 16), jnp.float32),
          pltpu.VMEM((B, d + 16), jnp.float32),
          pltpu.VMEM((208, d + 16), jnp.float32),
          pltpu.VMEM_SHARED((N_NODES, d + 16), jnp.float32),
          pltpu.SemaphoreType.DMA,
          pltpu.SemaphoreType.DMA,
          pltpu.SemaphoreType.DMA,
          pltpu.SemaphoreType.DMA,
      ],
  )
  def agg(h_hbm, asrc_hbm, adst_hbm, srcb_hbm, dstb_hbm,
          msg_out,
          asrc_v, adst_v, src_v, dst_v, wbuf0, wbuf1, rg0, rg1, rs0, rs1,
          zblk, acc_sh, sem_g0, sem_g1, sem_s0, sem_s1):
    cid = lax.axis_index("c")
    sid = lax.axis_index("s")
    wid = cid * NS + sid
    sb = BPT * wid + jnp.maximum(wid - (NW - 4), 0)
    has_extra = wid >= NW - 4

    pltpu.sync_copy(asrc_hbm, asrc_v)
    pltpu.sync_copy(adst_hbm, adst_v)
    pltpu.sync_copy(srcb_hbm.at[pl.ds(sb, BPT + 1)], src_v)
    pltpu.sync_copy(dstb_hbm.at[pl.ds(sb, BPT + 1)], dst_v)

    zv = jnp.zeros((16,), jnp.float32)
    lanes = lax.iota(jnp.int32, 16)

    def zero_body(i, _):
      for j in range(d // 16 + 1):
        zblk[i, pl.ds(j * 16, 16)] = zv
      return 0

    lax.fori_loop(0, 208, zero_body, 0)
    base = sid * RPT
    for j in range(RPT // 208):
      pltpu.sync_copy(zblk, acc_sh.at[pl.ds(base + j * 208, 208)])

    @pl.when(sid == NS - 1)
    def _():
      tail = NS * RPT
      pltpu.sync_copy(zblk.at[pl.ds(0, REM)], acc_sh.at[pl.ds(tail, REM)])

    plsc.subcore_barrier()

    def compute_w(b, wbuf):
      for k in range(B // 16):
        si = src_v[b, pl.ds(k * 16, 16)]
        di = dst_v[b, pl.ds(k * 16, 16)]
        t = plsc.load_gather(asrc_v, [si]) + plsc.load_gather(adst_v, [di])
        t = jnp.where(t >= 0.0, t, t * 0.2)
        wbuf[pl.ds(k * 16, 16)] = jnp.exp(t)

    lane0 = lanes == 0

    def scale(rg, rs, wbuf):
      for k in range(B // 16):
        wv = wbuf[pl.ds(k * 16, 16)]
        for i in range(16):
          e = k * 16 + i
          for j in range(d // 16):
            rs[e, pl.ds(j * 16, 16)] = rg[e, pl.ds(j * 16, 16)] * wv[i]
          rs[e, pl.ds(d, 16)] = jnp.where(lane0, wv[i], 0.0)

    def drain_gather(b, rg, sem):
      pltpu.make_async_copy(h_hbm.at[src_v.at[b]], rg, sem).wait()

    def drain_scatter(b, rs, sem):
      pltpu.make_async_copy(rs, acc_sh.at[dst_v.at[b]], sem).wait()

    pltpu.async_copy(h_hbm.at[src_v.at[0]], rg0, sem_g0)

    def pair_body(i, _):
      b0 = 2 * i
      b1 = b0 + 1
      compute_w(b0, wbuf0)

      @pl.when(i > 0)
      def _():
        drain_scatter(jnp.maximum(b1 - 2, 0), rs1, sem_s1)

      cp_g1 = pltpu.async_copy(h_hbm.at[src_v.at[b1]], rg1, sem_g1)
      drain_gather(b0, rg0, sem_g0)
      scale(rg0, rs0, wbuf0)
      cp_s0 = pltpu.async_copy(rs0, acc_sh.at[dst_v.at[b0]], sem_s0,
                               add=True)
      pltpu.async_copy(h_hbm.at[src_v.at[b0 + 2]], rg0, sem_g0)
      compute_w(b1, wbuf1)
      cp_g1.wait()
      scale(rg1, rs1, wbuf1)
      cp_s0.wait()
      pltpu.async_copy(rs1, acc_sh.at[dst_v.at[b1]], sem_s1, add=True)
      return 0

    lax.fori_loop(0, BPT // 2, pair_body, 0)

    drain_scatter(BPT - 1, rs1, sem_s1)
    compute_w(BPT, wbuf0)
    drain_gather(BPT, rg0, sem_g0)

    @pl.when(has_extra)
    def _():
      scale(rg0, rs0, wbuf0)
      pltpu.sync_copy(rs0, acc_sh.at[dst_v.at[BPT]], add=True)

    plsc.subcore_barrier()

    pltpu.sync_copy(acc_sh.at[pl.ds(base, RPT)],
                    msg_out.at[cid, pl.ds(base, RPT)])

    @pl.when(sid == NS - 1)
    def _():
      tail = NS * RPT
      pltpu.sync_copy(acc_sh.at[pl.ds(tail, REM)],
                      msg_out.at[cid, pl.ds(tail, REM)])

  return agg(h, a_src_n, a_dst_n, srcb, dstb)


def _dense_in(x, W1, a1s, a1d):
  def body(x_ref, w_ref, as_ref, ad_ref, h_ref, s_ref, d_ref):
    h = jnp.dot(x_ref[...], w_ref[...], preferred_element_type=jnp.float32)
    h_ref[...] = h
    s_ref[...] = jnp.dot(h, as_ref[...], preferred_element_type=jnp.float32)
    d_ref[...] = jnp.dot(h, ad_ref[...], preferred_element_type=jnp.float32)

  return pl.pallas_call(
      body,
      out_shape=[
          jax.ShapeDtypeStruct((N_NODES, D_HID), jnp.float32),
          jax.ShapeDtypeStruct((N_NODES, 1), jnp.float32),
          jax.ShapeDtypeStruct((N_NODES, 1), jnp.float32),
      ],
  )(x, W1, a1s.reshape(D_HID, 1), a1d.reshape(D_HID, 1))


def _combine_mid(msg, h, asrc, adst, b1, W2, a2s, a2d):
  def body(m_ref, h_ref, s_ref, dd_ref, b_ref, w_ref, as_ref, ad_ref,
           h2_ref, s2_ref, d2_ref):
    acc = m_ref[0] + m_ref[1]
    msgs = acc[:, :D_HID]
    den_e = acc[:, D_HID:D_HID + 1]
    t = s_ref[...] + dd_ref[...]
    t = jnp.where(t >= 0.0, t, t * 0.2)
    ws = jnp.exp(t)
    out = (msgs + ws * h_ref[...]) / (den_e + ws + 1e-16) + b_ref[...]
    out = jnp.where(out > 0.0, out, jnp.exp(out) - 1.0)
    h2 = jnp.dot(out, w_ref[...], preferred_element_type=jnp.float32)
    h2_ref[...] = h2
    s2_ref[...] = jnp.dot(h2, as_ref[...], preferred_element_type=jnp.float32)
    d2_ref[...] = jnp.dot(h2, ad_ref[...], preferred_element_type=jnp.float32)

  return pl.pallas_call(
      body,
      out_shape=[
          jax.ShapeDtypeStruct((N_NODES, D_OUT), jnp.float32),
          jax.ShapeDtypeStruct((N_NODES, 1), jnp.float32),
          jax.ShapeDtypeStruct((N_NODES, 1), jnp.float32),
      ],
  )(msg, h, asrc, adst, b1.reshape(1, D_HID), W2,
    a2s.reshape(D_OUT, 1), a2d.reshape(D_OUT, 1))


def _combine_out(msg, h, asrc, adst, b2):
  def body(m_ref, h_ref, s_ref, dd_ref, b_ref, o_ref):
    acc = m_ref[0] + m_ref[1]
    msgs = acc[:, :D_OUT]
    den_e = acc[:, D_OUT:D_OUT + 1]
    t = s_ref[...] + dd_ref[...]
    t = jnp.where(t >= 0.0, t, t * 0.2)
    ws = jnp.exp(t)
    o_ref[...] = (msgs + ws * h_ref[...]) / (den_e + ws + 1e-16) + b_ref[...]

  return pl.pallas_call(
      body,
      out_shape=jax.ShapeDtypeStruct((N_NODES, D_OUT), jnp.float32),
  )(msg, h, asrc, adst, b2.reshape(1, D_OUT))


def kernel(x, edge_index, W1, a1_src, a1_dst, b1, W2, a2_src, a2_dst, b2):
  ei = edge_index.astype(jnp.int32)
  srcb = ei[0].reshape(NB, B)
  dstb = ei[1].reshape(NB, B)

  h1, s1, d1 = _dense_in(x, W1, a1_src, a1_dst)
  msg1 = _edge_aggregate(h1, s1.reshape(N_NODES), d1.reshape(N_NODES),
                         srcb, dstb, D_HID)
  h2, s2, d2 = _combine_mid(msg1, h1, s1, d1, b1, W2, a2_src, a2_dst)
  msg2 = _edge_aggregate(h2, s2.reshape(N_NODES), d2.reshape(N_NODES),
                         srcb, dstb, D_OUT)
  return _combine_out(msg2, h2, s2, d2, b2)


# speedup vs baseline: 1.1031x; 1.1031x over previous
import functools

import jax
import jax.numpy as jnp
from jax import lax
from jax.experimental import pallas as pl
from jax.experimental.pallas import tpu as pltpu
from jax.experimental.pallas import tpu_sc as plsc

N_NODES = 10000
N_EDGES = 320000
D_IN = 128
D_HID = 32
D_OUT = 16

NC = 2
NS = 16
NW = NC * NS
B = 128
NB = N_EDGES // B
BPT = NB // NW
RPT = 624
REM = N_NODES - NS * RPT


def _edge_aggregate(h, a_src_n, a_dst_n, srcb, dstb, d):
  mesh = plsc.VectorSubcoreMesh(core_axis_name="c", subcore_axis_name="s",
                                num_cores=NC, num_subcores=NS)

  @functools.partial(
      pl.kernel,
      out_type=[
          jax.ShapeDtypeStruct((NC, N_NODES, d), jnp.float32),
          jax.ShapeDtypeStruct((NC, NS, N_NODES), jnp.float32),
      ],
      mesh=mesh,
      compiler_params=pltpu.CompilerParams(use_tc_tiling_on_sc=False,
                                           needs_layout_passes=False),
      scratch_types=[
          pltpu.VMEM((N_NODES,), jnp.float32),
          pltpu.VMEM((N_NODES,), jnp.float32),
          pltpu.VMEM((BPT + 1, B), jnp.int32),
          pltpu.VMEM((BPT + 1, B), jnp.int32),
          pltpu.VMEM((B,), jnp.float32),
          pltpu.VMEM((B,), jnp.float32),
          pltpu.VMEM((B, d), jnp.float32),
          pltpu.VMEM((B, d), jnp.float32),
          pltpu.VMEM((B, d), jnp.float32),
          pltpu.VMEM((B, d), jnp.float32),
          pltpu.VMEM((N_NODES,), jnp.float32),
          pltpu.VMEM((208, d), jnp.float32),
          pltpu.VMEM_SHARED((N_NODES, d), jnp.float32),
          pltpu.SemaphoreType.DMA,
          pltpu.SemaphoreType.DMA,
          pltpu.SemaphoreType.DMA,
          pltpu.SemaphoreType.DMA,
      ],
  )
  def agg(h_hbm, asrc_hbm, adst_hbm, srcb_hbm, dstb_hbm,
          msg_out, den_out,
          asrc_v, adst_v, src_v, dst_v, wbuf0, wbuf1, rg0, rg1, rs0, rs1,
          denom_v, zblk, acc_sh, sem_g0, sem_g1, sem_s0, sem_s1):
    cid = lax.axis_index("c")
    sid = lax.axis_index("s")
    wid = cid * NS + sid
    sb = BPT * wid + jnp.maximum(wid - (NW - 4), 0)
    has_extra = wid >= NW - 4

    pltpu.sync_copy(asrc_hbm, asrc_v)
    pltpu.sync_copy(adst_hbm, adst_v)
    pltpu.sync_copy(srcb_hbm.at[pl.ds(sb, BPT + 1)], src_v)
    pltpu.sync_copy(dstb_hbm.at[pl.ds(sb, BPT + 1)], dst_v)

    zv = jnp.zeros((16,), jnp.float32)

    def zero_body(i, _):
      for j in range(d // 16):
        zblk[i, pl.ds(j * 16, 16)] = zv
      return 0

    lax.fori_loop(0, 208, zero_body, 0)

    def zero_den(i, _):
      denom_v[pl.ds(i * 16, 16)] = zv
      return 0

    lax.fori_loop(0, N_NODES // 16, zero_den, 0)
    base = sid * RPT
    for j in range(RPT // 208):
      pltpu.sync_copy(zblk, acc_sh.at[pl.ds(base + j * 208, 208)])

    @pl.when(sid == NS - 1)
    def _():
      tail = NS * RPT
      pltpu.sync_copy(zblk.at[pl.ds(0, REM)], acc_sh.at[pl.ds(tail, REM)])

    plsc.subcore_barrier()

    def compute_w(b, wbuf):
      for k in range(B // 16):
        si = src_v[b, pl.ds(k * 16, 16)]
        di = dst_v[b, pl.ds(k * 16, 16)]
        t = plsc.load_gather(asrc_v, [si]) + plsc.load_gather(adst_v, [di])
        t = jnp.where(t >= 0.0, t, t * 0.2)
        w = jnp.exp(t)
        wbuf[pl.ds(k * 16, 16)] = w
        plsc.addupdate_scatter(denom_v, [di], w)

    def scale(rg, rs, wbuf):
      for k in range(B // 16):
        wv = wbuf[pl.ds(k * 16, 16)]
        for i in range(16):
          e = k * 16 + i
          for j in range(d // 16):
            rs[e, pl.ds(j * 16, 16)] = rg[e, pl.ds(j * 16, 16)] * wv[i]

    def drain_gather(b, rg, sem):
      pltpu.make_async_copy(h_hbm.at[src_v.at[b]], rg, sem).wait()

    def drain_scatter(b, rs, sem):
      pltpu.make_async_copy(rs, acc_sh.at[dst_v.at[b]], sem).wait()

    pltpu.async_copy(h_hbm.at[src_v.at[0]], rg0, sem_g0)

    def pair_body(i, _):
      b0 = 2 * i
      b1 = b0 + 1
      compute_w(b0, wbuf0)

      @pl.when(i > 0)
      def _():
        drain_scatter(jnp.maximum(b1 - 2, 0), rs1, sem_s1)

      cp_g1 = pltpu.async_copy(h_hbm.at[src_v.at[b1]], rg1, sem_g1)
      drain_gather(b0, rg0, sem_g0)
      scale(rg0, rs0, wbuf0)
      cp_s0 = pltpu.async_copy(rs0, acc_sh.at[dst_v.at[b0]], sem_s0,
                               add=True)
      pltpu.async_copy(h_hbm.at[src_v.at[b0 + 2]], rg0, sem_g0)
      compute_w(b1, wbuf1)
      cp_g1.wait()
      scale(rg1, rs1, wbuf1)
      cp_s0.wait()
      pltpu.async_copy(rs1, acc_sh.at[dst_v.at[b1]], sem_s1, add=True)
      return 0

    lax.fori_loop(0, BPT // 2, pair_body, 0)

    drain_scatter(BPT - 1, rs1, sem_s1)
    drain_gather(BPT, rg0, sem_g0)

    @pl.when(has_extra)
    def _():
      compute_w(BPT, wbuf0)
      scale(rg0, rs0, wbuf0)
      pltpu.sync_copy(rs0, acc_sh.at[dst_v.at[BPT]], add=True)

    pltpu.sync_copy(denom_v, den_out.at[cid, sid])
    plsc.subcore_barrier()

    pltpu.sync_copy(acc_sh.at[pl.ds(base, RPT)],
                    msg_out.at[cid, pl.ds(base, RPT)])

    @pl.when(sid == NS - 1)
    def _():
      tail = NS * RPT
      pltpu.sync_copy(acc_sh.at[pl.ds(tail, REM)],
                      msg_out.at[cid, pl.ds(tail, REM)])

  return agg(h, a_src_n, a_dst_n, srcb, dstb)


def _dense_in(x, W1, a1s, a1d):
  def body(x_ref, w_ref, as_ref, ad_ref, h_ref, s_ref, d_ref):
    h = jnp.dot(x_ref[...], w_ref[...], preferred_element_type=jnp.float32)
    h_ref[...] = h
    s_ref[...] = jnp.dot(h, as_ref[...], preferred_element_type=jnp.float32).T
    d_ref[...] = jnp.dot(h, ad_ref[...], preferred_element_type=jnp.float32).T

  return pl.pallas_call(
      body,
      out_shape=[
          jax.ShapeDtypeStruct((N_NODES, D_HID), jnp.float32),
          jax.ShapeDtypeStruct((1, N_NODES), jnp.float32),
          jax.ShapeDtypeStruct((1, N_NODES), jnp.float32),
      ],
  )(x, W1, a1s.reshape(D_HID, 1), a1d.reshape(D_HID, 1))


def _combine_mid(msg, den, h, asrc, adst, b1, W2, a2s, a2d):
  def body(m_ref, dn_ref, h_ref, s_ref, dd_ref, b_ref, w_ref, as_ref, ad_ref,
           h2_ref, s2_ref, d2_ref):
    msgs = m_ref[0] + m_ref[1]
    den_e = jnp.sum(dn_ref[0] + dn_ref[1], axis=0, keepdims=True).T
    t = s_ref[...] + dd_ref[...]
    t = jnp.where(t >= 0.0, t, t * 0.2)
    ws = jnp.exp(t).T
    out = (msgs + ws * h_ref[...]) / (den_e + ws + 1e-16) + b_ref[...]
    out = jnp.where(out > 0.0, out, jnp.exp(out) - 1.0)
    h2 = jnp.dot(out, w_ref[...], preferred_element_type=jnp.float32)
    h2_ref[...] = h2
    s2_ref[...] = jnp.dot(h2, as_ref[...],
                          preferred_element_type=jnp.float32).T
    d2_ref[...] = jnp.dot(h2, ad_ref[...],
                          preferred_element_type=jnp.float32).T

  return pl.pallas_call(
      body,
      out_shape=[
          jax.ShapeDtypeStruct((N_NODES, D_OUT), jnp.float32),
          jax.ShapeDtypeStruct((1, N_NODES), jnp.float32),
          jax.ShapeDtypeStruct((1, N_NODES), jnp.float32),
      ],
  )(msg, den, h, asrc, adst, b1.reshape(1, D_HID), W2,
    a2s.reshape(D_OUT, 1), a2d.reshape(D_OUT, 1))


def _combine_out(msg, den, h, asrc, adst, b2):
  def body(m_ref, dn_ref, h_ref, s_ref, dd_ref, b_ref, o_ref):
    msgs = m_ref[0] + m_ref[1]
    den_e = jnp.sum(dn_ref[0] + dn_ref[1], axis=0, keepdims=True).T
    t = s_ref[...] + dd_ref[...]
    t = jnp.where(t >= 0.0, t, t * 0.2)
    ws = jnp.exp(t).T
    o_ref[...] = (msgs + ws * h_ref[...]) / (den_e + ws + 1e-16) + b_ref[...]

  return pl.pallas_call(
      body,
      out_shape=jax.ShapeDtypeStruct((N_NODES, D_OUT), jnp.float32),
  )(msg, den, h, asrc, adst, b2.reshape(1, D_OUT))


def kernel(x, edge_index, W1, a1_src, a1_dst, b1, W2, a2_src, a2_dst, b2):
  ei = edge_index.astype(jnp.int32)
  srcb = ei[0].reshape(NB, B)
  dstb = ei[1].reshape(NB, B)

  h1, s1, d1 = _dense_in(x, W1, a1_src, a1_dst)
  msg1, den1 = _edge_aggregate(h1, s1.reshape(N_NODES), d1.reshape(N_NODES),
                               srcb, dstb, D_HID)
  h2, s2, d2 = _combine_mid(msg1, den1, h1, s1, d1, b1, W2, a2_src, a2_dst)
  msg2, den2 = _edge_aggregate(h2, s2.reshape(N_NODES), d2.reshape(N_NODES),
                               srcb, dstb, D_OUT)
  return _combine_out(msg2, den2, h2, s2, d2, b2)


# speedup vs baseline: 1.1102x; 1.0065x over previous
import functools

import jax
import jax.numpy as jnp
from jax import lax
from jax.experimental import pallas as pl
from jax.experimental.pallas import tpu as pltpu
from jax.experimental.pallas import tpu_sc as plsc

N_NODES = 10000
N_EDGES = 320000
D_IN = 128
D_HID = 32
D_OUT = 16

NC = 2
NS = 16
NW = NC * NS
B = 128
NB = N_EDGES // B
BPT = NB // NW
RPT = 624
REM = N_NODES - NS * RPT


def _edge_aggregate(h, a_src_n, a_dst_n, srcb, dstb, d):
  mesh = plsc.VectorSubcoreMesh(core_axis_name="c", subcore_axis_name="s",
                                num_cores=NC, num_subcores=NS)

  @functools.partial(
      pl.kernel,
      out_type=[
          jax.ShapeDtypeStruct((NC, N_NODES, d), jnp.float32),
          jax.ShapeDtypeStruct((NC, NS, N_NODES), jnp.float32),
      ],
      mesh=mesh,
      compiler_params=pltpu.CompilerParams(use_tc_tiling_on_sc=False,
                                           needs_layout_passes=False),
      scratch_types=[
          pltpu.VMEM((N_NODES,), jnp.float32),
          pltpu.VMEM((N_NODES,), jnp.float32),
          pltpu.VMEM((BPT + 1, B), jnp.int32),
          pltpu.VMEM((BPT + 1, B), jnp.int32),
          pltpu.VMEM((B,), jnp.float32),
          pltpu.VMEM((B,), jnp.float32),
          pltpu.VMEM((B, d), jnp.float32),
          pltpu.VMEM((B, d), jnp.float32),
          pltpu.VMEM((B, d), jnp.float32),
          pltpu.VMEM((B, d), jnp.float32),
          pltpu.VMEM((N_NODES,), jnp.float32),
          pltpu.VMEM((208, d), jnp.float32),
          pltpu.VMEM_SHARED((N_NODES, d), jnp.float32),
          pltpu.SemaphoreType.DMA,
          pltpu.SemaphoreType.DMA,
          pltpu.SemaphoreType.DMA,
          pltpu.SemaphoreType.DMA,
      ],
  )
  def agg(h_hbm, asrc_hbm, adst_hbm, srcb_hbm, dstb_hbm,
          msg_out, den_out,
          asrc_v, adst_v, src_v, dst_v, wbuf0, wbuf1, rg0, rg1, rs0, rs1,
          denom_v, zblk, acc_sh, sem_g0, sem_g1, sem_s0, sem_s1):
    cid = lax.axis_index("c")
    sid = lax.axis_index("s")
    wid = cid * NS + sid
    sb = BPT * wid + jnp.maximum(wid - (NW - 4), 0)
    has_extra = wid >= NW - 4

    pltpu.sync_copy(asrc_hbm, asrc_v)
    pltpu.sync_copy(adst_hbm, adst_v)
    pltpu.sync_copy(srcb_hbm.at[pl.ds(sb, BPT + 1)], src_v)
    pltpu.sync_copy(dstb_hbm.at[pl.ds(sb, BPT + 1)], dst_v)

    zv = jnp.zeros((16,), jnp.float32)

    def zero_body(i, _):
      for j in range(d // 16):
        zblk[i, pl.ds(j * 16, 16)] = zv
      return 0

    lax.fori_loop(0, 208, zero_body, 0)

    def zero_den(i, _):
      denom_v[pl.ds(i * 16, 16)] = zv
      return 0

    lax.fori_loop(0, N_NODES // 16, zero_den, 0)
    base = sid * RPT
    for j in range(RPT // 208):
      pltpu.sync_copy(zblk, acc_sh.at[pl.ds(base + j * 208, 208)])

    @pl.when(sid == NS - 1)
    def _():
      tail = NS * RPT
      pltpu.sync_copy(zblk.at[pl.ds(0, REM)], acc_sh.at[pl.ds(tail, REM)])

    plsc.subcore_barrier()

    def compute_w(b, wbuf):
      for k in range(B // 16):
        si = src_v[b, pl.ds(k * 16, 16)]
        di = dst_v[b, pl.ds(k * 16, 16)]
        t = plsc.load_gather(asrc_v, [si]) + plsc.load_gather(adst_v, [di])
        t = jnp.where(t >= 0.0, t, t * 0.2)
        w = jnp.exp(t)
        wbuf[pl.ds(k * 16, 16)] = w
        plsc.addupdate_scatter(denom_v, [di], w)

    def scale(rg, rs, wbuf):
      for k in range(B // 16):
        wv = wbuf[pl.ds(k * 16, 16)]
        for i in range(16):
          e = k * 16 + i
          for j in range(d // 16):
            rs[e, pl.ds(j * 16, 16)] = rg[e, pl.ds(j * 16, 16)] * wv[i]

    def drain_gather(b, rg, sem):
      pltpu.make_async_copy(h_hbm.at[src_v.at[b]], rg, sem).wait()

    def drain_scatter(b, rs, sem):
      pltpu.make_async_copy(rs, acc_sh.at[dst_v.at[b]], sem).wait()

    pltpu.async_copy(h_hbm.at[src_v.at[0]], rg0, sem_g0)

    def pair_body(i, _):
      b0 = 2 * i
      b1 = b0 + 1
      compute_w(b0, wbuf0)
      cp_g1 = pltpu.async_copy(h_hbm.at[src_v.at[b1]], rg1, sem_g1)

      @pl.when(i > 0)
      def _():
        drain_scatter(jnp.maximum(b0 - 2, 0), rs0, sem_s0)

      drain_gather(b0, rg0, sem_g0)
      scale(rg0, rs0, wbuf0)
      pltpu.async_copy(rs0, acc_sh.at[dst_v.at[b0]], sem_s0, add=True)
      pltpu.async_copy(h_hbm.at[src_v.at[b0 + 2]], rg0, sem_g0)
      compute_w(b1, wbuf1)

      @pl.when(i > 0)
      def _():
        drain_scatter(jnp.maximum(b1 - 2, 0), rs1, sem_s1)

      cp_g1.wait()
      scale(rg1, rs1, wbuf1)
      pltpu.async_copy(rs1, acc_sh.at[dst_v.at[b1]], sem_s1, add=True)
      return 0

    lax.fori_loop(0, BPT // 2, pair_body, 0)

    drain_scatter(BPT - 2, rs0, sem_s0)
    drain_scatter(BPT - 1, rs1, sem_s1)
    drain_gather(BPT, rg0, sem_g0)

    @pl.when(has_extra)
    def _():
      compute_w(BPT, wbuf0)
      scale(rg0, rs0, wbuf0)
      pltpu.sync_copy(rs0, acc_sh.at[dst_v.at[BPT]], add=True)

    pltpu.sync_copy(denom_v, den_out.at[cid, sid])
    plsc.subcore_barrier()

    pltpu.sync_copy(acc_sh.at[pl.ds(base, RPT)],
                    msg_out.at[cid, pl.ds(base, RPT)])

    @pl.when(sid == NS - 1)
    def _():
      tail = NS * RPT
      pltpu.sync_copy(acc_sh.at[pl.ds(tail, REM)],
                      msg_out.at[cid, pl.ds(tail, REM)])

  return agg(h, a_src_n, a_dst_n, srcb, dstb)


def _dense_in(x, W1, a1s, a1d):
  def body(x_ref, w_ref, as_ref, ad_ref, h_ref, s_ref, d_ref):
    h = jnp.dot(x_ref[...], w_ref[...], preferred_element_type=jnp.float32)
    h_ref[...] = h
    s_ref[...] = jnp.dot(h, as_ref[...], preferred_element_type=jnp.float32).T
    d_ref[...] = jnp.dot(h, ad_ref[...], preferred_element_type=jnp.float32).T

  return pl.pallas_call(
      body,
      out_shape=[
          jax.ShapeDtypeStruct((N_NODES, D_HID), jnp.float32),
          jax.ShapeDtypeStruct((1, N_NODES), jnp.float32),
          jax.ShapeDtypeStruct((1, N_NODES), jnp.float32),
      ],
  )(x, W1, a1s.reshape(D_HID, 1), a1d.reshape(D_HID, 1))


def _combine_mid(msg, den, h, asrc, adst, b1, W2, a2s, a2d):
  def body(m_ref, dn_ref, h_ref, s_ref, dd_ref, b_ref, w_ref, as_ref, ad_ref,
           h2_ref, s2_ref, d2_ref):
    msgs = m_ref[0] + m_ref[1]
    den_e = jnp.sum(dn_ref[0] + dn_ref[1], axis=0, keepdims=True).T
    t = s_ref[...] + dd_ref[...]
    t = jnp.where(t >= 0.0, t, t * 0.2)
    ws = jnp.exp(t).T
    out = (msgs + ws * h_ref[...]) / (den_e + ws + 1e-16) + b_ref[...]
    out = jnp.where(out > 0.0, out, jnp.exp(out) - 1.0)
    h2 = jnp.dot(out, w_ref[...], preferred_element_type=jnp.float32)
    h2_ref[...] = h2
    s2_ref[...] = jnp.dot(h2, as_ref[...],
                          preferred_element_type=jnp.float32).T
    d2_ref[...] = jnp.dot(h2, ad_ref[...],
                          preferred_element_type=jnp.float32).T

  return pl.pallas_call(
      body,
      out_shape=[
          jax.ShapeDtypeStruct((N_NODES, D_OUT), jnp.float32),
          jax.ShapeDtypeStruct((1, N_NODES), jnp.float32),
          jax.ShapeDtypeStruct((1, N_NODES), jnp.float32),
      ],
  )(msg, den, h, asrc, adst, b1.reshape(1, D_HID), W2,
    a2s.reshape(D_OUT, 1), a2d.reshape(D_OUT, 1))


def _combine_out(msg, den, h, asrc, adst, b2):
  def body(m_ref, dn_ref, h_ref, s_ref, dd_ref, b_ref, o_ref):
    msgs = m_ref[0] + m_ref[1]
    den_e = jnp.sum(dn_ref[0] + dn_ref[1], axis=0, keepdims=True).T
    t = s_ref[...] + dd_ref[...]
    t = jnp.where(t >= 0.0, t, t * 0.2)
    ws = jnp.exp(t).T
    o_ref[...] = (msgs + ws * h_ref[...]) / (den_e + ws + 1e-16) + b_ref[...]

  return pl.pallas_call(
      body,
      out_shape=jax.ShapeDtypeStruct((N_NODES, D_OUT), jnp.float32),
  )(msg, den, h, asrc, adst, b2.reshape(1, D_OUT))


def kernel(x, edge_index, W1, a1_src, a1_dst, b1, W2, a2_src, a2_dst, b2):
  ei = edge_index.astype(jnp.int32)
  srcb = ei[0].reshape(NB, B)
  dstb = ei[1].reshape(NB, B)

  h1, s1, d1 = _dense_in(x, W1, a1_src, a1_dst)
  msg1, den1 = _edge_aggregate(h1, s1.reshape(N_NODES), d1.reshape(N_NODES),
                               srcb, dstb, D_HID)
  h2, s2, d2 = _combine_mid(msg1, den1, h1, s1, d1, b1, W2, a2_src, a2_dst)
  msg2, den2 = _edge_aggregate(h2, s2.reshape(N_NODES), d2.reshape(N_NODES),
                               srcb, dstb, D_OUT)
  return _combine_out(msg2, den2, h2, s2, d2, b2)


# speedup vs baseline: 1.1162x; 1.0054x over previous
import functools

import jax
import jax.numpy as jnp
import numpy as np
from jax import lax
from jax.experimental import pallas as pl
from jax.experimental.pallas import tpu as pltpu
from jax.experimental.pallas import tpu_sc as plsc

N_NODES = 10000
N_EDGES = 320000
D_IN = 128
D_HID = 32
D_OUT = 16

_PERM_HID = np.concatenate([np.arange(0, 32, 2), np.arange(1, 32, 2)])

NC = 2
NS = 16
NW = NC * NS
B = 128
NB = N_EDGES // B
BPT = NB // NW
RPT = 624
REM = N_NODES - NS * RPT


def _edge_aggregate(h, a_src_n, a_dst_n, srcb, dstb, d, bf16_h=False):
  mesh = plsc.VectorSubcoreMesh(core_axis_name="c", subcore_axis_name="s",
                                num_cores=NC, num_subcores=NS)

  @functools.partial(
      pl.kernel,
      out_type=[
          jax.ShapeDtypeStruct((NC, N_NODES, d), jnp.float32),
          jax.ShapeDtypeStruct((NC, NS, N_NODES), jnp.float32),
      ],
      mesh=mesh,
      compiler_params=pltpu.CompilerParams(use_tc_tiling_on_sc=False,
                                           needs_layout_passes=False),
      scratch_types=[
          pltpu.VMEM((N_NODES,), jnp.float32),
          pltpu.VMEM((N_NODES,), jnp.float32),
          pltpu.VMEM((BPT + 1, B), jnp.int32),
          pltpu.VMEM((BPT + 1, B), jnp.int32),
          pltpu.VMEM((B,), jnp.float32),
          pltpu.VMEM((B,), jnp.float32),
          pltpu.VMEM((B, d), jnp.bfloat16 if bf16_h else jnp.float32),
          pltpu.VMEM((B, d), jnp.bfloat16 if bf16_h else jnp.float32),
          pltpu.VMEM((B, d), jnp.float32),
          pltpu.VMEM((B, d), jnp.float32),
          pltpu.VMEM((N_NODES,), jnp.float32),
          pltpu.VMEM((208, d), jnp.float32),
          pltpu.VMEM_SHARED((N_NODES, d), jnp.float32),
          pltpu.SemaphoreType.DMA,
          pltpu.SemaphoreType.DMA,
          pltpu.SemaphoreType.DMA,
          pltpu.SemaphoreType.DMA,
      ],
  )
  def agg(h_hbm, asrc_hbm, adst_hbm, srcb_hbm, dstb_hbm,
          msg_out, den_out,
          asrc_v, adst_v, src_v, dst_v, wbuf0, wbuf1, rg0, rg1, rs0, rs1,
          denom_v, zblk, acc_sh, sem_g0, sem_g1, sem_s0, sem_s1):
    cid = lax.axis_index("c")
    sid = lax.axis_index("s")
    wid = cid * NS + sid
    sb = BPT * wid + jnp.maximum(wid - (NW - 4), 0)
    has_extra = wid >= NW - 4

    pltpu.sync_copy(asrc_hbm, asrc_v)
    pltpu.sync_copy(adst_hbm, adst_v)
    pltpu.sync_copy(srcb_hbm.at[pl.ds(sb, BPT + 1)], src_v)
    pltpu.sync_copy(dstb_hbm.at[pl.ds(sb, BPT + 1)], dst_v)

    zv = jnp.zeros((16,), jnp.float32)

    def zero_body(i, _):
      for j in range(d // 16):
        zblk[i, pl.ds(j * 16, 16)] = zv
      return 0

    lax.fori_loop(0, 208, zero_body, 0)

    def zero_den(i, _):
      denom_v[pl.ds(i * 16, 16)] = zv
      return 0

    lax.fori_loop(0, N_NODES // 16, zero_den, 0)
    base = sid * RPT
    for j in range(RPT // 208):
      pltpu.sync_copy(zblk, acc_sh.at[pl.ds(base + j * 208, 208)])

    @pl.when(sid == NS - 1)
    def _():
      tail = NS * RPT
      pltpu.sync_copy(zblk.at[pl.ds(0, REM)], acc_sh.at[pl.ds(tail, REM)])

    plsc.subcore_barrier()

    def compute_w(b, wbuf):
      for k in range(B // 16):
        si = src_v[b, pl.ds(k * 16, 16)]
        di = dst_v[b, pl.ds(k * 16, 16)]
        t = plsc.load_gather(asrc_v, [si]) + plsc.load_gather(adst_v, [di])
        t = jnp.where(t >= 0.0, t, t * 0.2)
        w = jnp.exp(t)
        wbuf[pl.ds(k * 16, 16)] = w
        plsc.addupdate_scatter(denom_v, [di], w)

    if bf16_h:
      def scale(rg, rs, wbuf):
        for k in range(B // 16):
          wv = wbuf[pl.ds(k * 16, 16)]
          for i in range(16):
            e = k * 16 + i
            ev, od = plsc.unpack(rg[e, pl.ds(0, 32)],
                                 format=plsc.PackFormat.INTERLEAVED)
            rs[e, pl.ds(0, 16)] = ev * wv[i]
            rs[e, pl.ds(16, 16)] = od * wv[i]
    else:
      def scale(rg, rs, wbuf):
        for k in range(B // 16):
          wv = wbuf[pl.ds(k * 16, 16)]
          for i in range(16):
            e = k * 16 + i
            for j in range(d // 16):
              rs[e, pl.ds(j * 16, 16)] = rg[e, pl.ds(j * 16, 16)] * wv[i]

    def drain_gather(b, rg, sem):
      pltpu.make_async_copy(h_hbm.at[src_v.at[b]], rg, sem).wait()

    def drain_scatter(b, rs, sem):
      pltpu.make_async_copy(rs, acc_sh.at[dst_v.at[b]], sem).wait()

    pltpu.async_copy(h_hbm.at[src_v.at[0]], rg0, sem_g0)

    def pair_body(i, _):
      b0 = 2 * i
      b1 = b0 + 1
      compute_w(b0, wbuf0)
      cp_g1 = pltpu.async_copy(h_hbm.at[src_v.at[b1]], rg1, sem_g1)

      @pl.when(i > 0)
      def _():
        drain_scatter(jnp.maximum(b0 - 2, 0), rs0, sem_s0)

      drain_gather(b0, rg0, sem_g0)
      scale(rg0, rs0, wbuf0)
      pltpu.async_copy(rs0, acc_sh.at[dst_v.at[b0]], sem_s0, add=True)
      pltpu.async_copy(h_hbm.at[src_v.at[b0 + 2]], rg0, sem_g0)
      compute_w(b1, wbuf1)

      @pl.when(i > 0)
      def _():
        drain_scatter(jnp.maximum(b1 - 2, 0), rs1, sem_s1)

      cp_g1.wait()
      scale(rg1, rs1, wbuf1)
      pltpu.async_copy(rs1, acc_sh.at[dst_v.at[b1]], sem_s1, add=True)
      return 0

    lax.fori_loop(0, BPT // 2, pair_body, 0)

    drain_scatter(BPT - 2, rs0, sem_s0)
    drain_scatter(BPT - 1, rs1, sem_s1)
    drain_gather(BPT, rg0, sem_g0)

    @pl.when(has_extra)
    def _():
      compute_w(BPT, wbuf0)
      scale(rg0, rs0, wbuf0)
      pltpu.sync_copy(rs0, acc_sh.at[dst_v.at[BPT]], add=True)

    pltpu.sync_copy(denom_v, den_out.at[cid, sid])
    plsc.subcore_barrier()

    pltpu.sync_copy(acc_sh.at[pl.ds(base, RPT)],
                    msg_out.at[cid, pl.ds(base, RPT)])

    @pl.when(sid == NS - 1)
    def _():
      tail = NS * RPT
      pltpu.sync_copy(acc_sh.at[pl.ds(tail, REM)],
                      msg_out.at[cid, pl.ds(tail, REM)])

  return agg(h, a_src_n, a_dst_n, srcb, dstb)


def _dense_in(x, W1, a1s, a1d):
  def body(x_ref, w_ref, wp_ref, as_ref, ad_ref,
           hp_ref, hb_ref, s_ref, d_ref):
    h = jnp.dot(x_ref[...], w_ref[...], preferred_element_type=jnp.float32)
    hb_ref[...] = h.astype(jnp.bfloat16)
    hp_ref[...] = jnp.dot(x_ref[...], wp_ref[...],
                          preferred_element_type=jnp.float32)
    s_ref[...] = jnp.dot(h, as_ref[...], preferred_element_type=jnp.float32).T
    d_ref[...] = jnp.dot(h, ad_ref[...], preferred_element_type=jnp.float32).T

  W1p = W1[:, _PERM_HID]
  return pl.pallas_call(
      body,
      out_shape=[
          jax.ShapeDtypeStruct((N_NODES, D_HID), jnp.float32),
          jax.ShapeDtypeStruct((N_NODES, D_HID), jnp.bfloat16),
          jax.ShapeDtypeStruct((1, N_NODES), jnp.float32),
          jax.ShapeDtypeStruct((1, N_NODES), jnp.float32),
      ],
  )(x, W1, W1p, a1s.reshape(D_HID, 1), a1d.reshape(D_HID, 1))


def _combine_mid(msg, den, h, asrc, adst, b1, W2, a2s, a2d):
  def body(m_ref, dn_ref, h_ref, s_ref, dd_ref, b_ref, w_ref, as_ref, ad_ref,
           h2_ref, s2_ref, d2_ref):
    msgs = m_ref[0] + m_ref[1]
    den_e = jnp.sum(dn_ref[0] + dn_ref[1], axis=0, keepdims=True).T
    t = s_ref[...] + dd_ref[...]
    t = jnp.where(t >= 0.0, t, t * 0.2)
    ws = jnp.exp(t).T
    out = (msgs + ws * h_ref[...]) / (den_e + ws + 1e-16) + b_ref[...]
    out = jnp.where(out > 0.0, out, jnp.exp(out) - 1.0)
    h2 = jnp.dot(out, w_ref[...], preferred_element_type=jnp.float32)
    h2_ref[...] = h2
    s2_ref[...] = jnp.dot(h2, as_ref[...],
                          preferred_element_type=jnp.float32).T
    d2_ref[...] = jnp.dot(h2, ad_ref[...],
                          preferred_element_type=jnp.float32).T

  return pl.pallas_call(
      body,
      out_shape=[
          jax.ShapeDtypeStruct((N_NODES, D_OUT), jnp.float32),
          jax.ShapeDtypeStruct((1, N_NODES), jnp.float32),
          jax.ShapeDtypeStruct((1, N_NODES), jnp.float32),
      ],
  )(msg, den, h, asrc, adst, b1.reshape(1, D_HID), W2,
    a2s.reshape(D_OUT, 1), a2d.reshape(D_OUT, 1))


def _combine_out(msg, den, h, asrc, adst, b2):
  def body(m_ref, dn_ref, h_ref, s_ref, dd_ref, b_ref, o_ref):
    msgs = m_ref[0] + m_ref[1]
    den_e = jnp.sum(dn_ref[0] + dn_ref[1], axis=0, keepdims=True).T
    t = s_ref[...] + dd_ref[...]
    t = jnp.where(t >= 0.0, t, t * 0.2)
    ws = jnp.exp(t).T
    o_ref[...] = (msgs + ws * h_ref[...]) / (den_e + ws + 1e-16) + b_ref[...]

  return pl.pallas_call(
      body,
      out_shape=jax.ShapeDtypeStruct((N_NODES, D_OUT), jnp.float32),
  )(msg, den, h, asrc, adst, b2.reshape(1, D_OUT))


def kernel(x, edge_index, W1, a1_src, a1_dst, b1, W2, a2_src, a2_dst, b2):
  ei = edge_index.astype(jnp.int32)
  srcb = ei[0].reshape(NB, B)
  dstb = ei[1].reshape(NB, B)

  h1p, h1b, s1, d1 = _dense_in(x, W1, a1_src, a1_dst)
  msg1, den1 = _edge_aggregate(h1b, s1.reshape(N_NODES), d1.reshape(N_NODES),
                               srcb, dstb, D_HID, bf16_h=True)
  h2, s2, d2 = _combine_mid(msg1, den1, h1p, s1, d1, b1[_PERM_HID],
                            W2[_PERM_HID, :], a2_src, a2_dst)
  msg2, den2 = _edge_aggregate(h2, s2.reshape(N_NODES), d2.reshape(N_NODES),
                               srcb, dstb, D_OUT)
  return _combine_out(msg2, den2, h2, s2, d2, b2)
